# Initial kernel scaffold; baseline (speedup 1.0000x reference)
import jax
import jax.numpy as jnp
from jax.experimental import pallas as pl


def kernel(x, edge_index, edge_attr, batch, node_W, node_b, edge_W, edge_b, lin1_W, lin1_b, bn_g, bn_b, lin2_W, lin2_b, ln_g, ln_b, out_W, out_b, p1_W, p1_b, p2_W, p2_b):
    raise NotImplementedError("write your pallas kernel here")


# speedup vs baseline: 2.6629x; 2.6629x over previous
import functools

import jax
import jax.numpy as jnp
from jax import lax
from jax.experimental import pallas as pl
from jax.experimental.pallas import tpu as pltpu
from jax.experimental.pallas import tpu_sc as plsc

EPS_GIN = 0.1
BN_EPS = 1e-5
LN_EPS = 1e-5

NC = 2
NS = 16
CHUNK = 80


def _sc_agg(src, dst, h, h_edge, n_pad):
  e_total = src.shape[0]
  hd = h.shape[1]
  ept = e_total // (NC * NS)
  nchunk = ept // CHUNK
  rows_per_tile = n_pad // NS

  mesh = plsc.VectorSubcoreMesh(core_axis_name="c", subcore_axis_name="s")

  @functools.partial(
      pl.kernel,
      out_type=jax.ShapeDtypeStruct((NC, n_pad, hd), jnp.float32),
      mesh=mesh,
      scratch_types=[
          pltpu.VMEM((CHUNK,), jnp.int32),
          pltpu.VMEM((CHUNK,), jnp.int32),
          pltpu.VMEM((CHUNK, hd), jnp.float32),
          pltpu.VMEM((CHUNK, hd), jnp.float32),
          pltpu.VMEM_SHARED((n_pad, hd), jnp.float32),
          pltpu.SemaphoreType.DMA,
      ],
  )
  def agg_kernel(src_hbm, dst_hbm, h_hbm, he_hbm, out_hbm,
                 sidx, didx, hrows, erows, aggsh, sem):
    c = lax.axis_index("c")
    s = lax.axis_index("s")

    def zfill(i, _):
      for j in range(hd // 16):
        hrows[i, pl.ds(j * 16, 16)] = jnp.zeros((16,), jnp.float32)
      return _
    lax.fori_loop(0, CHUNK, zfill, None)

    def zcopy(k, _):
      pltpu.sync_copy(hrows, aggsh.at[pl.ds(s * rows_per_tile + k * CHUNK,
                                            CHUNK)])
      return _
    lax.fori_loop(0, rows_per_tile // CHUNK, zcopy, None)
    plsc.subcore_barrier()

    base_e = (c * NS + s) * ept

    def chunk_body(k, _):
      off = base_e + k * CHUNK
      pltpu.sync_copy(src_hbm.at[pl.ds(off, CHUNK)], sidx)
      pltpu.sync_copy(dst_hbm.at[pl.ds(off, CHUNK)], didx)
      pltpu.async_copy(h_hbm.at[sidx], hrows, sem).wait()
      pltpu.sync_copy(he_hbm.at[pl.ds(off, CHUNK)], erows)

      def row_body(i, _):
        for j in range(hd // 16):
          sl = pl.ds(j * 16, 16)
          hrows[i, sl] = jnp.maximum(hrows[i, sl] + erows[i, sl], 0.0)
        return _
      lax.fori_loop(0, CHUNK, row_body, None)

      pltpu.sync_copy(hrows, aggsh.at[didx], add=True)
      return _
    lax.fori_loop(0, nchunk, chunk_body, None)
    plsc.subcore_barrier()

    pltpu.sync_copy(aggsh.at[pl.ds(s * rows_per_tile, rows_per_tile)],
                    out_hbm.at[c].at[pl.ds(s * rows_per_tile, rows_per_tile)])

  return agg_kernel(src, dst, h, h_edge)


def _encode_kernel(x_ref, w_ref, b_ref, o_ref):
  o_ref[...] = jnp.maximum(
      jnp.dot(x_ref[...], w_ref[...], preferred_element_type=jnp.float32)
      + b_ref[...], 0.0)


def _encode(x, w_t, b, blk):
  n, _ = x.shape
  h = w_t.shape[1]
  return pl.pallas_call(
      _encode_kernel,
      grid=(n // blk,),
      in_specs=[
          pl.BlockSpec((blk, x.shape[1]), lambda i: (i, 0)),
          pl.BlockSpec(w_t.shape, lambda i: (0, 0)),
          pl.BlockSpec(b.shape, lambda i: (0, 0)),
      ],
      out_specs=pl.BlockSpec((blk, h), lambda i: (i, 0)),
      out_shape=jax.ShapeDtypeStruct((n, h), jnp.float32),
  )(x, w_t, b)


def _layer_kernel(h_ref, a0_ref, a1_ref, w1_ref, b1_ref, w2_ref, b2_ref,
                  o_ref):
  z = (1.0 + EPS_GIN) * h_ref[...] + a0_ref[...] + a1_ref[...]
  z = jnp.maximum(
      jnp.dot(z, w1_ref[...], preferred_element_type=jnp.float32)
      + b1_ref[...], 0.0)
  o_ref[...] = jnp.maximum(
      jnp.dot(z, w2_ref[...], preferred_element_type=jnp.float32)
      + b2_ref[...], 0.0)


def _layer_mlp(h, a0, a1, w1_t, b1, w2_t, b2, blk):
  n, hd = h.shape
  full = lambda a: pl.BlockSpec(a.shape, lambda i: (0, 0))
  rows = pl.BlockSpec((blk, hd), lambda i: (i, 0))
  return pl.pallas_call(
      _layer_kernel,
      grid=(n // blk,),
      in_specs=[rows, rows, rows, full(w1_t), full(b1), full(w2_t), full(b2)],
      out_specs=rows,
      out_shape=jax.ShapeDtypeStruct((n, hd), jnp.float32),
  )(h, a0, a1, w1_t, b1, w2_t, b2)


def _final_kernel(h_ref, batch_ref, lng_ref, lnb_ref, ow_ref, ob_ref,
                  p1w_ref, p1b_ref, p2w_ref, p2b_ref,
                  ho_ref, perf_ref, sums_ref, cnts_ref, *, num_groups,
                  nblocks):
  i = pl.program_id(0)
  hv = h_ref[...]
  mu = jnp.mean(hv, axis=1, keepdims=True)
  var = jnp.mean(jnp.square(hv), axis=1, keepdims=True) - jnp.square(mu)
  hn = (hv - mu) * lax.rsqrt(var + LN_EPS) * lng_ref[...] + lnb_ref[...]
  ho = jnp.maximum(
      jnp.dot(hn, ow_ref[...], preferred_element_type=jnp.float32)
      + ob_ref[...], 0.0)
  ho_ref[...] = ho

  bb = batch_ref[0]
  gi = lax.broadcasted_iota(jnp.int32, (num_groups, bb.shape[1]), 0)
  oh = (gi == bb).astype(jnp.float32)

  @pl.when(i == 0)
  def _():
    sums_ref[...] = jnp.zeros_like(sums_ref)
    cnts_ref[...] = jnp.zeros_like(cnts_ref)

  sums_ref[...] += jnp.dot(oh, ho, preferred_element_type=jnp.float32)
  cnts_ref[...] += jnp.sum(oh, axis=1, keepdims=True)

  @pl.when(i == nblocks - 1)
  def _():
    emb = sums_ref[...] / jnp.maximum(cnts_ref[...], 1.0)
    p = jnp.maximum(
        jnp.dot(emb, p1w_ref[...], preferred_element_type=jnp.float32)
        + p1b_ref[...], 0.0)
    perf_ref[...] = (jnp.dot(p, p2w_ref[...],
                             preferred_element_type=jnp.float32)
                     + p2b_ref[...])


def _final(h, batch2d, ln_g, ln_b, ow_t, ob, p1w_t, p1b, p2w_t, p2b, blk):
  n, hd = h.shape
  out_d = ow_t.shape[1]
  num_groups = 64
  nblocks = n // blk
  full = lambda a: pl.BlockSpec(a.shape, lambda i: (0, 0))
  kern = functools.partial(_final_kernel, num_groups=num_groups,
                           nblocks=nblocks)
  return pl.pallas_call(
      kern,
      grid=(nblocks,),
      in_specs=[
          pl.BlockSpec((blk, hd), lambda i: (i, 0)),
          pl.BlockSpec((1, 1, blk), lambda i: (i, 0, 0)),
          full(ln_g), full(ln_b), full(ow_t), full(ob),
          full(p1w_t), full(p1b), full(p2w_t), full(p2b),
      ],
      out_specs=[
          pl.BlockSpec((blk, out_d), lambda i: (i, 0)),
          pl.BlockSpec((num_groups, 1), lambda i: (0, 0)),
      ],
      out_shape=[
          jax.ShapeDtypeStruct((n, out_d), jnp.float32),
          jax.ShapeDtypeStruct((num_groups, 1), jnp.float32),
      ],
      scratch_shapes=[
          pltpu.VMEM((num_groups, out_d), jnp.float32),
          pltpu.VMEM((num_groups, 1), jnp.float32),
      ],
  )(h, batch2d, ln_g, ln_b, ow_t, ob, p1w_t, p1b, p2w_t, p2b)


def kernel(x, edge_index, edge_attr, batch, node_W, node_b, edge_W, edge_b,
           lin1_W, lin1_b, bn_g, bn_b, lin2_W, lin2_b, ln_g, ln_b,
           out_W, out_b, p1_W, p1_b, p2_W, p2_b):
  n, d_node = x.shape
  e = edge_attr.shape[0]
  hd = node_W.shape[0]
  num_layers = lin1_W.shape[0]
  num_groups = 64
  out_d = out_W.shape[0]

  blk = 2048
  n_pad = ((n + blk - 1) // blk) * blk

  x_p = jnp.pad(x, ((0, n_pad - n), (0, 0)))
  batch_p = jnp.pad(batch, (0, n_pad - n), constant_values=num_groups)
  batch2d = batch_p.reshape(n_pad // blk, 1, blk)

  src = edge_index[0]
  dst = edge_index[1]

  h = _encode(x_p, node_W.T, node_b.reshape(1, hd), blk)
  h_edge = _encode(edge_attr, edge_W.T, edge_b.reshape(1, hd), 2560)

  bn_scale = bn_g / jnp.sqrt(1.0 + BN_EPS)
  eff_w1 = jnp.transpose(lin1_W, (0, 2, 1)) * bn_scale[:, None, :]
  eff_b1 = lin1_b * bn_scale + bn_b

  for l in range(num_layers):
    agg = _sc_agg(src, dst, h, h_edge, n_pad)
    h = _layer_mlp(h, agg[0], agg[1],
                   eff_w1[l], eff_b1[l].reshape(1, hd),
                   lin2_W[l].T, lin2_b[l].reshape(1, hd), blk)

  h_out, perf = _final(h, batch2d, ln_g.reshape(1, hd), ln_b.reshape(1, hd),
                       out_W.T, out_b.reshape(1, out_d),
                       p1_W.T, p1_b.reshape(1, p1_b.shape[0]),
                       p2_W.T, p2_b.reshape(1, 1), blk)
  return h_out[:n], perf.reshape(num_groups)


# speedup vs baseline: 5.0845x; 1.9094x over previous
import functools

import jax
import jax.numpy as jnp
from jax import lax
from jax.experimental import pallas as pl
from jax.experimental.pallas import tpu as pltpu
from jax.experimental.pallas import tpu_sc as plsc

EPS_GIN = 0.1
BN_EPS = 1e-5
LN_EPS = 1e-5

NC = 2
NS = 16
CHUNK = 80


def _sc_agg(src, dst, h, h_edge, n_pad, nchunk):
  hd = h.shape[1]
  ept = nchunk * CHUNK
  rows_per_tile = n_pad // NS
  assert nchunk % 2 == 1

  mesh = plsc.VectorSubcoreMesh(core_axis_name="c", subcore_axis_name="s")

  @functools.partial(
      pl.kernel,
      out_type=jax.ShapeDtypeStruct((NC, n_pad, hd), jnp.float32),
      mesh=mesh,
      scratch_types=[
          pltpu.VMEM((CHUNK,), jnp.int32),
          pltpu.VMEM((CHUNK,), jnp.int32),
          pltpu.VMEM((CHUNK,), jnp.int32),
          pltpu.VMEM((CHUNK,), jnp.int32),
          pltpu.VMEM((CHUNK, hd), jnp.float32),
          pltpu.VMEM((CHUNK, hd), jnp.float32),
          pltpu.VMEM((CHUNK, hd), jnp.float32),
          pltpu.VMEM((CHUNK, hd), jnp.float32),
          pltpu.VMEM_SHARED((n_pad, hd), jnp.float32),
          pltpu.SemaphoreType.DMA,
          pltpu.SemaphoreType.DMA,
          pltpu.SemaphoreType.DMA,
          pltpu.SemaphoreType.DMA,
          pltpu.SemaphoreType.DMA,
          pltpu.SemaphoreType.DMA,
      ],
  )
  def agg_kernel(src_hbm, dst_hbm, h_hbm, he_hbm, out_hbm,
                 sidx0, sidx1, didx0, didx1, hrows0, hrows1, erows0, erows1,
                 aggsh, isem0, isem1, gsem0, gsem1, esem0, esem1):
    c = lax.axis_index("c")
    s = lax.axis_index("s")
    wid = c * NS + s
    base_e = wid * ept

    bufs = ((sidx0, didx0, hrows0, erows0, isem0, gsem0, esem0),
            (sidx1, didx1, hrows1, erows1, isem1, gsem1, esem1))

    def issue_idx(k, sidx, didx, isem):
      pltpu.async_copy(src_hbm.at[pl.ds(base_e + k * CHUNK, CHUNK)], sidx,
                       isem)
      pltpu.async_copy(dst_hbm.at[pl.ds(base_e + k * CHUNK, CHUNK)], didx,
                       isem)

    def wait_idx(k, sidx, didx, isem):
      pltpu.make_async_copy(src_hbm.at[pl.ds(base_e + k * CHUNK, CHUNK)],
                            sidx, isem).wait()
      pltpu.make_async_copy(dst_hbm.at[pl.ds(base_e + k * CHUNK, CHUNK)],
                            didx, isem).wait()

    def issue_rows(k, sidx, hrows, erows, gsem, esem):
      pltpu.async_copy(h_hbm.at[sidx], hrows, gsem)
      pltpu.async_copy(he_hbm.at[pl.ds(base_e + k * CHUNK, CHUNK)], erows,
                       esem)

    def wait_rows(k, sidx, hrows, erows, gsem, esem):
      pltpu.make_async_copy(h_hbm.at[sidx], hrows, gsem).wait()
      pltpu.make_async_copy(he_hbm.at[pl.ds(base_e + k * CHUNK, CHUNK)],
                            erows, esem).wait()

    issue_idx(0, sidx0, didx0, isem0)

    def zfill(i, _):
      for j in range(hd // 16):
        hrows0[i, pl.ds(j * 16, 16)] = jnp.zeros((16,), jnp.float32)
      return _
    lax.fori_loop(0, CHUNK, zfill, None)

    def zcopy(k, _):
      pltpu.sync_copy(hrows0, aggsh.at[pl.ds(s * rows_per_tile + k * CHUNK,
                                             CHUNK)])
      return _
    lax.fori_loop(0, rows_per_tile // CHUNK, zcopy, None)

    wait_idx(0, sidx0, didx0, isem0)
    issue_rows(0, sidx0, hrows0, erows0, gsem0, esem0)
    issue_idx(1, sidx1, didx1, isem1)
    plsc.subcore_barrier()

    def step(k, cur, oth):
      sidx, didx, hrows, erows, isem, gsem, esem = cur
      osidx, odidx, ohrows, oerows, oisem, ogsem, oesem = oth
      nk = jnp.minimum(k + 1, nchunk - 1)
      nk2 = jnp.minimum(k + 2, nchunk - 1)
      wait_rows(k, sidx, hrows, erows, gsem, esem)
      wait_idx(nk, osidx, odidx, oisem)
      issue_rows(nk, osidx, ohrows, oerows, ogsem, oesem)

      def row_body(i, _):
        for j in range(hd // 16):
          sl = pl.ds(j * 16, 16)
          hrows[i, sl] = jnp.maximum(hrows[i, sl] + erows[i, sl], 0.0)
        return _
      lax.fori_loop(0, CHUNK, row_body, None)

      pltpu.sync_copy(hrows, aggsh.at[didx], add=True)
      issue_idx(nk2, sidx, didx, isem)

    def chunk_iter(k, _):
      @pl.when(k % 2 == 0)
      def _():
        step(k, bufs[0], bufs[1])

      @pl.when(k % 2 == 1)
      def _():
        step(k, bufs[1], bufs[0])
      return _
    lax.fori_loop(0, nchunk, chunk_iter, None)

    wait_rows(nchunk - 1, sidx1, hrows1, erows1, gsem1, esem1)
    wait_idx(nchunk - 1, sidx0, didx0, isem0)
    plsc.subcore_barrier()

    pltpu.sync_copy(aggsh.at[pl.ds(s * rows_per_tile, rows_per_tile)],
                    out_hbm.at[c].at[pl.ds(s * rows_per_tile, rows_per_tile)])

  return agg_kernel(src, dst, h, h_edge)


def _encode_kernel(x_ref, w_ref, b_ref, o_ref):
  o_ref[...] = jnp.maximum(
      jnp.dot(x_ref[...], w_ref[...], preferred_element_type=jnp.float32)
      + b_ref[...], 0.0)


def _encode(x, w_t, b, blk):
  n, _ = x.shape
  h = w_t.shape[1]
  return pl.pallas_call(
      _encode_kernel,
      grid=(n // blk,),
      in_specs=[
          pl.BlockSpec((blk, x.shape[1]), lambda i: (i, 0)),
          pl.BlockSpec(w_t.shape, lambda i: (0, 0)),
          pl.BlockSpec(b.shape, lambda i: (0, 0)),
      ],
      out_specs=pl.BlockSpec((blk, h), lambda i: (i, 0)),
      out_shape=jax.ShapeDtypeStruct((n, h), jnp.float32),
  )(x, w_t, b)


def _layer_kernel(h_ref, a0_ref, a1_ref, w1_ref, b1_ref, w2_ref, b2_ref,
                  o_ref):
  z = (1.0 + EPS_GIN) * h_ref[...] + a0_ref[...] + a1_ref[...]
  z = jnp.maximum(
      jnp.dot(z, w1_ref[...], preferred_element_type=jnp.float32)
      + b1_ref[...], 0.0)
  o_ref[...] = jnp.maximum(
      jnp.dot(z, w2_ref[...], preferred_element_type=jnp.float32)
      + b2_ref[...], 0.0)


def _layer_mlp(h, a0, a1, w1_t, b1, w2_t, b2, blk):
  n, hd = h.shape
  full = lambda a: pl.BlockSpec(a.shape, lambda i: (0, 0))
  rows = pl.BlockSpec((blk, hd), lambda i: (i, 0))
  return pl.pallas_call(
      _layer_kernel,
      grid=(n // blk,),
      in_specs=[rows, rows, rows, full(w1_t), full(b1), full(w2_t), full(b2)],
      out_specs=rows,
      out_shape=jax.ShapeDtypeStruct((n, hd), jnp.float32),
  )(h, a0, a1, w1_t, b1, w2_t, b2)


def _final_kernel(h_ref, batch_ref, lng_ref, lnb_ref, ow_ref, ob_ref,
                  p1w_ref, p1b_ref, p2w_ref, p2b_ref,
                  ho_ref, perf_ref, sums_ref, cnts_ref, *, num_groups,
                  nblocks):
  i = pl.program_id(0)
  hv = h_ref[...]
  mu = jnp.mean(hv, axis=1, keepdims=True)
  var = jnp.mean(jnp.square(hv), axis=1, keepdims=True) - jnp.square(mu)
  hn = (hv - mu) * lax.rsqrt(var + LN_EPS) * lng_ref[...] + lnb_ref[...]
  ho = jnp.maximum(
      jnp.dot(hn, ow_ref[...], preferred_element_type=jnp.float32)
      + ob_ref[...], 0.0)
  ho_ref[...] = ho

  bb = batch_ref[0]
  gi = lax.broadcasted_iota(jnp.int32, (num_groups, bb.shape[1]), 0)
  oh = (gi == bb).astype(jnp.float32)

  @pl.when(i == 0)
  def _():
    sums_ref[...] = jnp.zeros_like(sums_ref)
    cnts_ref[...] = jnp.zeros_like(cnts_ref)

  sums_ref[...] += jnp.dot(oh, ho, preferred_element_type=jnp.float32)
  cnts_ref[...] += jnp.sum(oh, axis=1, keepdims=True)

  @pl.when(i == nblocks - 1)
  def _():
    emb = sums_ref[...] / jnp.maximum(cnts_ref[...], 1.0)
    p = jnp.maximum(
        jnp.dot(emb, p1w_ref[...], preferred_element_type=jnp.float32)
        + p1b_ref[...], 0.0)
    perf_ref[...] = (jnp.dot(p, p2w_ref[...],
                             preferred_element_type=jnp.float32)
                     + p2b_ref[...])


def _final(h, batch2d, ln_g, ln_b, ow_t, ob, p1w_t, p1b, p2w_t, p2b, blk):
  n, hd = h.shape
  out_d = ow_t.shape[1]
  num_groups = 64
  nblocks = n // blk
  full = lambda a: pl.BlockSpec(a.shape, lambda i: (0, 0))
  kern = functools.partial(_final_kernel, num_groups=num_groups,
                           nblocks=nblocks)
  return pl.pallas_call(
      kern,
      grid=(nblocks,),
      in_specs=[
          pl.BlockSpec((blk, hd), lambda i: (i, 0)),
          pl.BlockSpec((1, 1, blk), lambda i: (i, 0, 0)),
          full(ln_g), full(ln_b), full(ow_t), full(ob),
          full(p1w_t), full(p1b), full(p2w_t), full(p2b),
      ],
      out_specs=[
          pl.BlockSpec((blk, out_d), lambda i: (i, 0)),
          pl.BlockSpec((num_groups, 1), lambda i: (0, 0)),
      ],
      out_shape=[
          jax.ShapeDtypeStruct((n, out_d), jnp.float32),
          jax.ShapeDtypeStruct((num_groups, 1), jnp.float32),
      ],
      scratch_shapes=[
          pltpu.VMEM((num_groups, out_d), jnp.float32),
          pltpu.VMEM((num_groups, 1), jnp.float32),
      ],
  )(h, batch2d, ln_g, ln_b, ow_t, ob, p1w_t, p1b, p2w_t, p2b)


def kernel(x, edge_index, edge_attr, batch, node_W, node_b, edge_W, edge_b,
           lin1_W, lin1_b, bn_g, bn_b, lin2_W, lin2_b, ln_g, ln_b,
           out_W, out_b, p1_W, p1_b, p2_W, p2_b):
  n, d_node = x.shape
  e = edge_attr.shape[0]
  hd = node_W.shape[0]
  num_layers = lin1_W.shape[0]
  num_groups = 64
  out_d = out_W.shape[0]

  blk = 2048
  n_pad = ((n + blk - 1) // blk) * blk

  x_p = jnp.pad(x, ((0, n_pad - n), (0, 0)))
  batch_p = jnp.pad(batch, (0, n_pad - n), constant_values=num_groups)
  batch2d = batch_p.reshape(n_pad // blk, 1, blk)

  nchunk = e // (NC * NS) // CHUNK
  src = edge_index[0]
  dst = edge_index[1]

  h = _encode(x_p, node_W.T, node_b.reshape(1, hd), blk)
  h_edge = _encode(edge_attr, edge_W.T, edge_b.reshape(1, hd), 2560)

  bn_scale = bn_g / jnp.sqrt(1.0 + BN_EPS)
  eff_w1 = jnp.transpose(lin1_W, (0, 2, 1)) * bn_scale[:, None, :]
  eff_b1 = lin1_b * bn_scale + bn_b

  for l in range(num_layers):
    agg = _sc_agg(src, dst, h, h_edge, n_pad, nchunk)
    h = _layer_mlp(h, agg[0], agg[1],
                   eff_w1[l], eff_b1[l].reshape(1, hd),
                   lin2_W[l].T, lin2_b[l].reshape(1, hd), blk)

  h_out, perf = _final(h, batch2d, ln_g.reshape(1, hd), ln_b.reshape(1, hd),
                       out_W.T, out_b.reshape(1, out_d),
                       p1_W.T, p1_b.reshape(1, p1_b.shape[0]),
                       p2_W.T, p2_b.reshape(1, 1), blk)
  return h_out[:n], perf.reshape(num_groups)


# speedup vs baseline: 5.3072x; 1.0438x over previous
import functools

import jax
import jax.numpy as jnp
from jax import lax
from jax.experimental import pallas as pl
from jax.experimental.pallas import tpu as pltpu
from jax.experimental.pallas import tpu_sc as plsc

EPS_GIN = 0.1
BN_EPS = 1e-5
LN_EPS = 1e-5

NC = 2
NS = 16
CHUNK = 80


def _sc_agg(src, dst, h, h_edge, n_pad, nchunk):
  hd = h.shape[1]
  ept = nchunk * CHUNK
  rows_per_tile = n_pad // NS
  assert nchunk % 2 == 1

  mesh = plsc.VectorSubcoreMesh(core_axis_name="c", subcore_axis_name="s")

  @functools.partial(
      pl.kernel,
      out_type=jax.ShapeDtypeStruct((NC, n_pad, hd), jnp.float32),
      mesh=mesh,
      scratch_types=[
          pltpu.VMEM((CHUNK,), jnp.int32),
          pltpu.VMEM((CHUNK,), jnp.int32),
          pltpu.VMEM((CHUNK,), jnp.int32),
          pltpu.VMEM((CHUNK,), jnp.int32),
          pltpu.VMEM((CHUNK, hd), jnp.float32),
          pltpu.VMEM((CHUNK, hd), jnp.float32),
          pltpu.VMEM((CHUNK, hd), jnp.float32),
          pltpu.VMEM((CHUNK, hd), jnp.float32),
          pltpu.VMEM((CHUNK,), jnp.int32),
          pltpu.VMEM((CHUNK,), jnp.int32),
          pltpu.VMEM_SHARED((n_pad, hd), jnp.float32),
          pltpu.SemaphoreType.DMA,
          pltpu.SemaphoreType.DMA,
          pltpu.SemaphoreType.DMA,
          pltpu.SemaphoreType.DMA,
          pltpu.SemaphoreType.DMA,
          pltpu.SemaphoreType.DMA,
          pltpu.SemaphoreType.DMA,
          pltpu.SemaphoreType.DMA,
      ],
  )
  def agg_kernel(src_hbm, dst_hbm, h_hbm, he_hbm, out_hbm,
                 sidx0, sidx1, didx0, didx1, hrows0, hrows1, erows0, erows1,
                 dscat0, dscat1, aggsh, isem0, isem1, gsem0, gsem1,
                 esem0, esem1, ssem0, ssem1):
    c = lax.axis_index("c")
    s = lax.axis_index("s")
    wid = c * NS + s
    base_e = wid * ept

    bufs = ((sidx0, didx0, hrows0, erows0, dscat0, isem0, gsem0, esem0,
             ssem0),
            (sidx1, didx1, hrows1, erows1, dscat1, isem1, gsem1, esem1,
             ssem1))

    def issue_idx(k, sidx, didx, isem):
      pltpu.async_copy(src_hbm.at[pl.ds(base_e + k * CHUNK, CHUNK)], sidx,
                       isem)
      pltpu.async_copy(dst_hbm.at[pl.ds(base_e + k * CHUNK, CHUNK)], didx,
                       isem)

    def wait_idx(k, sidx, didx, isem):
      pltpu.make_async_copy(src_hbm.at[pl.ds(base_e + k * CHUNK, CHUNK)],
                            sidx, isem).wait()
      pltpu.make_async_copy(dst_hbm.at[pl.ds(base_e + k * CHUNK, CHUNK)],
                            didx, isem).wait()

    def issue_rows(k, sidx, hrows, erows, gsem, esem):
      pltpu.async_copy(h_hbm.at[sidx], hrows, gsem)
      pltpu.async_copy(he_hbm.at[pl.ds(base_e + k * CHUNK, CHUNK)], erows,
                       esem)

    def wait_rows(k, sidx, hrows, erows, gsem, esem):
      pltpu.make_async_copy(h_hbm.at[sidx], hrows, gsem).wait()
      pltpu.make_async_copy(he_hbm.at[pl.ds(base_e + k * CHUNK, CHUNK)],
                            erows, esem).wait()

    issue_idx(0, sidx0, didx0, isem0)

    def zfill(i, _):
      for j in range(hd // 16):
        hrows0[i, pl.ds(j * 16, 16)] = jnp.zeros((16,), jnp.float32)
      return _
    lax.fori_loop(0, CHUNK, zfill, None)

    def zcopy(k, _):
      pltpu.sync_copy(hrows0, aggsh.at[pl.ds(s * rows_per_tile + k * CHUNK,
                                             CHUNK)])
      return _
    lax.fori_loop(0, rows_per_tile // CHUNK, zcopy, None)

    wait_idx(0, sidx0, didx0, isem0)
    issue_rows(0, sidx0, hrows0, erows0, gsem0, esem0)
    issue_idx(1, sidx1, didx1, isem1)
    plsc.subcore_barrier()

    def step(k, cur, oth):
      sidx, didx, hrows, erows, dscat, isem, gsem, esem, ssem = cur
      (osidx, odidx, ohrows, oerows, odscat, oisem, ogsem, oesem,
       ossem) = oth
      nk = jnp.minimum(k + 1, nchunk - 1)
      nk2 = jnp.minimum(k + 2, nchunk - 1)
      wait_rows(k, sidx, hrows, erows, gsem, esem)
      wait_idx(nk, osidx, odidx, oisem)

      @pl.when(k >= 1)
      def _():
        pltpu.make_async_copy(ohrows, aggsh.at[odscat], ossem).wait()
      issue_rows(nk, osidx, ohrows, oerows, ogsem, oesem)

      for j in range(CHUNK // 16):
        dscat[pl.ds(j * 16, 16)] = didx[pl.ds(j * 16, 16)]

      def row_body(i, _):
        for j in range(hd // 16):
          sl = pl.ds(j * 16, 16)
          hrows[i, sl] = jnp.maximum(hrows[i, sl] + erows[i, sl], 0.0)
        return _
      lax.fori_loop(0, CHUNK, row_body, None)

      pltpu.async_copy(hrows, aggsh.at[dscat], ssem, add=True)
      issue_idx(nk2, sidx, didx, isem)

    def chunk_iter(k, _):
      @pl.when(k % 2 == 0)
      def _():
        step(k, bufs[0], bufs[1])

      @pl.when(k % 2 == 1)
      def _():
        step(k, bufs[1], bufs[0])
      return _
    lax.fori_loop(0, nchunk, chunk_iter, None)

    wait_rows(nchunk - 1, sidx1, hrows1, erows1, gsem1, esem1)
    wait_idx(nchunk - 1, sidx0, didx0, isem0)
    pltpu.make_async_copy(hrows0, aggsh.at[dscat0], ssem0).wait()
    plsc.subcore_barrier()

    pltpu.sync_copy(aggsh.at[pl.ds(s * rows_per_tile, rows_per_tile)],
                    out_hbm.at[c].at[pl.ds(s * rows_per_tile, rows_per_tile)])

  return agg_kernel(src, dst, h, h_edge)


def _encode_kernel(x_ref, w_ref, b_ref, o_ref):
  o_ref[...] = jnp.maximum(
      jnp.dot(x_ref[...], w_ref[...], preferred_element_type=jnp.float32)
      + b_ref[...], 0.0)


def _encode(x, w_t, b, blk):
  n, _ = x.shape
  h = w_t.shape[1]
  return pl.pallas_call(
      _encode_kernel,
      grid=(n // blk,),
      in_specs=[
          pl.BlockSpec((blk, x.shape[1]), lambda i: (i, 0)),
          pl.BlockSpec(w_t.shape, lambda i: (0, 0)),
          pl.BlockSpec(b.shape, lambda i: (0, 0)),
      ],
      out_specs=pl.BlockSpec((blk, h), lambda i: (i, 0)),
      out_shape=jax.ShapeDtypeStruct((n, h), jnp.float32),
  )(x, w_t, b)


def _layer_kernel(h_ref, a0_ref, a1_ref, w1_ref, b1_ref, w2_ref, b2_ref,
                  o_ref):
  z = (1.0 + EPS_GIN) * h_ref[...] + a0_ref[...] + a1_ref[...]
  z = jnp.maximum(
      jnp.dot(z, w1_ref[...], preferred_element_type=jnp.float32)
      + b1_ref[...], 0.0)
  o_ref[...] = jnp.maximum(
      jnp.dot(z, w2_ref[...], preferred_element_type=jnp.float32)
      + b2_ref[...], 0.0)


def _layer_mlp(h, a0, a1, w1_t, b1, w2_t, b2, blk):
  n, hd = h.shape
  full = lambda a: pl.BlockSpec(a.shape, lambda i: (0, 0))
  rows = pl.BlockSpec((blk, hd), lambda i: (i, 0))
  return pl.pallas_call(
      _layer_kernel,
      grid=(n // blk,),
      in_specs=[rows, rows, rows, full(w1_t), full(b1), full(w2_t), full(b2)],
      out_specs=rows,
      out_shape=jax.ShapeDtypeStruct((n, hd), jnp.float32),
  )(h, a0, a1, w1_t, b1, w2_t, b2)


def _final_kernel(h_ref, batch_ref, lng_ref, lnb_ref, ow_ref, ob_ref,
                  p1w_ref, p1b_ref, p2w_ref, p2b_ref,
                  ho_ref, perf_ref, sums_ref, cnts_ref, *, num_groups,
                  nblocks):
  i = pl.program_id(0)
  hv = h_ref[...]
  mu = jnp.mean(hv, axis=1, keepdims=True)
  var = jnp.mean(jnp.square(hv), axis=1, keepdims=True) - jnp.square(mu)
  hn = (hv - mu) * lax.rsqrt(var + LN_EPS) * lng_ref[...] + lnb_ref[...]
  ho = jnp.maximum(
      jnp.dot(hn, ow_ref[...], preferred_element_type=jnp.float32)
      + ob_ref[...], 0.0)
  ho_ref[...] = ho

  bb = batch_ref[0]
  gi = lax.broadcasted_iota(jnp.int32, (num_groups, bb.shape[1]), 0)
  oh = (gi == bb).astype(jnp.float32)

  @pl.when(i == 0)
  def _():
    sums_ref[...] = jnp.zeros_like(sums_ref)
    cnts_ref[...] = jnp.zeros_like(cnts_ref)

  sums_ref[...] += jnp.dot(oh, ho, preferred_element_type=jnp.float32)
  cnts_ref[...] += jnp.sum(oh, axis=1, keepdims=True)

  @pl.when(i == nblocks - 1)
  def _():
    emb = sums_ref[...] / jnp.maximum(cnts_ref[...], 1.0)
    p = jnp.maximum(
        jnp.dot(emb, p1w_ref[...], preferred_element_type=jnp.float32)
        + p1b_ref[...], 0.0)
    perf_ref[...] = (jnp.dot(p, p2w_ref[...],
                             preferred_element_type=jnp.float32)
                     + p2b_ref[...])


def _final(h, batch2d, ln_g, ln_b, ow_t, ob, p1w_t, p1b, p2w_t, p2b, blk):
  n, hd = h.shape
  out_d = ow_t.shape[1]
  num_groups = 64
  nblocks = n // blk
  full = lambda a: pl.BlockSpec(a.shape, lambda i: (0, 0))
  kern = functools.partial(_final_kernel, num_groups=num_groups,
                           nblocks=nblocks)
  return pl.pallas_call(
      kern,
      grid=(nblocks,),
      in_specs=[
          pl.BlockSpec((blk, hd), lambda i: (i, 0)),
          pl.BlockSpec((1, 1, blk), lambda i: (i, 0, 0)),
          full(ln_g), full(ln_b), full(ow_t), full(ob),
          full(p1w_t), full(p1b), full(p2w_t), full(p2b),
      ],
      out_specs=[
          pl.BlockSpec((blk, out_d), lambda i: (i, 0)),
          pl.BlockSpec((num_groups, 1), lambda i: (0, 0)),
      ],
      out_shape=[
          jax.ShapeDtypeStruct((n, out_d), jnp.float32),
          jax.ShapeDtypeStruct((num_groups, 1), jnp.float32),
      ],
      scratch_shapes=[
          pltpu.VMEM((num_groups, out_d), jnp.float32),
          pltpu.VMEM((num_groups, 1), jnp.float32),
      ],
  )(h, batch2d, ln_g, ln_b, ow_t, ob, p1w_t, p1b, p2w_t, p2b)


def kernel(x, edge_index, edge_attr, batch, node_W, node_b, edge_W, edge_b,
           lin1_W, lin1_b, bn_g, bn_b, lin2_W, lin2_b, ln_g, ln_b,
           out_W, out_b, p1_W, p1_b, p2_W, p2_b):
  n, d_node = x.shape
  e = edge_attr.shape[0]
  hd = node_W.shape[0]
  num_layers = lin1_W.shape[0]
  num_groups = 64
  out_d = out_W.shape[0]

  blk = 2048
  n_pad = ((n + blk - 1) // blk) * blk

  x_p = jnp.pad(x, ((0, n_pad - n), (0, 0)))
  batch_p = jnp.pad(batch, (0, n_pad - n), constant_values=num_groups)
  batch2d = batch_p.reshape(n_pad // blk, 1, blk)

  nchunk = e // (NC * NS) // CHUNK
  src = edge_index[0]
  dst = edge_index[1]

  h = _encode(x_p, node_W.T, node_b.reshape(1, hd), blk)
  h_edge = _encode(edge_attr, edge_W.T, edge_b.reshape(1, hd), 2560)

  bn_scale = bn_g / jnp.sqrt(1.0 + BN_EPS)
  eff_w1 = jnp.transpose(lin1_W, (0, 2, 1)) * bn_scale[:, None, :]
  eff_b1 = lin1_b * bn_scale + bn_b

  for l in range(num_layers):
    agg = _sc_agg(src, dst, h, h_edge, n_pad, nchunk)
    h = _layer_mlp(h, agg[0], agg[1],
                   eff_w1[l], eff_b1[l].reshape(1, hd),
                   lin2_W[l].T, lin2_b[l].reshape(1, hd), blk)

  h_out, perf = _final(h, batch2d, ln_g.reshape(1, hd), ln_b.reshape(1, hd),
                       out_W.T, out_b.reshape(1, out_d),
                       p1_W.T, p1_b.reshape(1, p1_b.shape[0]),
                       p2_W.T, p2_b.reshape(1, 1), blk)
  return h_out[:n], perf.reshape(num_groups)
